# baseline (device time: 168943 ns/iter reference)
import jax
import jax.numpy as jnp
from jax import lax
from jax.experimental import pallas as pl
from jax.experimental.pallas import tpu as pltpu

N_DEV = 8
G_RS = 4
AG_DIMS = ((1, 2, 4), (2, 4, 1))
SGN = (1, -1)


def kernel(x, w_mat):
    M, K = x.shape
    _, N = w_mat.shape
    CH = M // N_DEV
    NL = N // 2
    NQ = NL // G_RS
    H = N_DEV - 1

    def body(x_ref, w_ref, out_ref, comm_r, comm_l, qag_r, qag_l, amax_buf,
             send_r, recv_r, send_l, recv_l,
             ag_send_r, ag_recv_r, ag_send_l, ag_recv_l,
             amax_send, amax_recv, credit_r, credit_l):
        my = lax.axis_index("i")
        left = (my - 1) % N_DEV
        right = (my + 1) % N_DEV

        barrier = pltpu.get_barrier_semaphore()
        for nbr in (left, right):
            pl.semaphore_signal(
                barrier, inc=1, device_id=(nbr,),
                device_id_type=pl.DeviceIdType.MESH,
            )
        pl.semaphore_wait(barrier, 2)

        def chunk(i):
            return pl.ds(pl.multiple_of(i * CH, CH), CH)

        cols = (pl.ds(0, NL), pl.ds(NL, NL))
        base = (0, NL)
        comms = (comm_r, comm_l)
        qags = (qag_r, qag_l)
        ag_ss = (ag_send_r, ag_send_l)
        ag_rs = (ag_recv_r, ag_recv_l)
        ssems = (send_r, send_l)
        rsems = (recv_r, recv_l)
        creds = (credit_r, credit_l)

        def tgt_of(X):
            return right if X == 0 else left

        def upstream_of(X):
            return left if X == 0 else right

        def rs_cols(X, g):
            return pl.ds(base[X] + g * NQ, NQ)

        def gemm_chunk(c):
            out_ref[chunk(c), :] = jnp.dot(
                x_ref[chunk(c), :], w_ref[...],
                preferred_element_type=jnp.float32,
            ).astype(out_ref.dtype)

        def mk_rs(h, X, g):
            slot = h % 2
            cs = (my - h) % N_DEV if X == 0 else (my + h) % N_DEV
            return pltpu.make_async_remote_copy(
                src_ref=out_ref.at[chunk(cs), rs_cols(X, g)],
                dst_ref=comms[X].at[slot, g],
                send_sem=ssems[X].at[slot, g],
                recv_sem=rsems[X].at[slot, g],
                device_id=(tgt_of(X),),
                device_id_type=pl.DeviceIdType.MESH,
            )

        gemm_chunk(my)

        desc = {}
        for g in range(G_RS):
            for X in (0, 1):
                d = mk_rs(0, X, g)
                d.start()
                desc[(X, g, 0)] = d

        for h in range(H):
            slot = h % 2
            if h < 3:
                gemm_chunk((my - h - 1) % N_DEV)
                gemm_chunk((my + h + 1) % N_DEV)
            elif h == 3:
                gemm_chunk((my + 4) % N_DEV)

            for g in range(G_RS):
                for X in (0, 1):
                    cr = (my - h - 1) % N_DEV if X == 0 else (my + h + 1) % N_DEV
                    sub = rs_cols(X, g)
                    d = desc[(X, g, slot)]
                    d.wait_recv()
                    out_ref[chunk(cr), sub] = (
                        out_ref[chunk(cr), sub].astype(jnp.float32)
                        + comms[X][slot, g].astype(jnp.float32)
                    ).astype(out_ref.dtype)
                    if h + 1 < H:
                        ns = (h + 1) % 2
                        if h >= 1:
                            desc[(X, g, ns)].wait_send()
                            pl.semaphore_wait(creds[X].at[ns, g], 1)
                        nd = mk_rs(h + 1, X, g)
                        nd.start()
                        desc[(X, g, ns)] = nd
                    pl.semaphore_signal(
                        creds[X].at[slot, g], inc=1,
                        device_id=(upstream_of(X),),
                        device_id_type=pl.DeviceIdType.MESH,
                    )

        for g in range(G_RS):
            for X in (0, 1):
                for s in (0, 1):
                    desc[(X, g, s)].wait_send()

        own = ((my + 1) % N_DEV, (my - 1) % N_DEV)

        local_amax = jnp.maximum(
            jnp.max(jnp.abs(out_ref[chunk(own[0]), cols[0]]).astype(jnp.float32)),
            jnp.max(jnp.abs(out_ref[chunk(own[1]), cols[1]]).astype(jnp.float32)),
        )
        amax_buf[pl.ds(my, 1), :] = jnp.full((1, 128), local_amax, jnp.float32)
        amax_rdmas = []
        for o in range(1, N_DEV):
            r = pltpu.make_async_remote_copy(
                src_ref=amax_buf.at[pl.ds(my, 1), :],
                dst_ref=amax_buf.at[pl.ds(my, 1), :],
                send_sem=amax_send.at[o],
                recv_sem=amax_recv.at[o],
                device_id=((my + o) % N_DEV,),
                device_id_type=pl.DeviceIdType.MESH,
            )
            r.start()
            amax_rdmas.append(r)
        for r in amax_rdmas:
            r.wait()
        amax = jnp.max(amax_buf[...])
        scale = amax / 127.0
        inv = 127.0 / amax

        def quantize(v_bf16):
            q = jnp.clip(jnp.round(v_bf16.astype(jnp.float32) * inv),
                         -127.0, 127.0)
            return q.astype(jnp.int8)

        def dequant(q_i8):
            return (q_i8.astype(jnp.float32) * scale).astype(out_ref.dtype)

        qag_r[chunk(own[0]), :] = quantize(out_ref[chunk(own[0]), cols[0]])
        qag_l[chunk(own[1]), :] = quantize(out_ref[chunk(own[1]), cols[1]])

        def spans_of(D):
            return ([0], [0, D[0]], [0, D[0], D[1], D[0] ^ D[1]])

        def held_chunk(X, s):
            return ((my ^ s) + SGN[X]) % N_DEV

        for k in range(3):
            step_ds = []
            for X in (0, 1):
                D = AG_DIMS[X]
                partner = my ^ D[k]
                for j, s in enumerate(spans_of(D)[k]):
                    c = held_chunk(X, s)
                    d = pltpu.make_async_remote_copy(
                        src_ref=qags[X].at[chunk(c), :],
                        dst_ref=qags[X].at[chunk(c), :],
                        send_sem=ag_ss[X].at[k, j],
                        recv_sem=ag_rs[X].at[k, j],
                        device_id=(partner,),
                        device_id_type=pl.DeviceIdType.MESH,
                    )
                    d.start()
                    step_ds.append((X, s, d))
            if k == 0:
                out_ref[chunk(own[0]), cols[0]] = dequant(qag_r[chunk(own[0]), :])
                out_ref[chunk(own[1]), cols[1]] = dequant(qag_l[chunk(own[1]), :])
            else:
                for X in (0, 1):
                    D = AG_DIMS[X]
                    for s in spans_of(D)[k - 1]:
                        c = held_chunk(X, D[k - 1] ^ s)
                        out_ref[chunk(c), cols[X]] = dequant(qags[X][chunk(c), :])
            if k < 2:
                for _, _, d in step_ds:
                    d.wait_recv()
                for _, _, d in step_ds:
                    d.wait_send()
            else:
                for X, s, d in step_ds:
                    d.wait_recv()
                    c = held_chunk(X, AG_DIMS[X][2] ^ s)
                    out_ref[chunk(c), cols[X]] = dequant(qags[X][chunk(c), :])
                for _, _, d in step_ds:
                    d.wait_send()

        for X in (0, 1):
            for g in range(G_RS):
                pl.semaphore_wait(creds[X].at[0, g], 1)
                pl.semaphore_wait(creds[X].at[1, g], 1)

    return pl.pallas_call(
        body,
        out_shape=jax.ShapeDtypeStruct((M, N), jnp.bfloat16),
        in_specs=[
            pl.BlockSpec(memory_space=pltpu.VMEM),
            pl.BlockSpec(memory_space=pltpu.VMEM),
        ],
        out_specs=pl.BlockSpec(memory_space=pltpu.VMEM),
        scratch_shapes=[
            pltpu.VMEM((2, G_RS, CH, NQ), jnp.bfloat16),
            pltpu.VMEM((2, G_RS, CH, NQ), jnp.bfloat16),
            pltpu.VMEM((M, NL), jnp.int8),
            pltpu.VMEM((M, NL), jnp.int8),
            pltpu.VMEM((N_DEV, 128), jnp.float32),
            pltpu.SemaphoreType.DMA((2, G_RS)),
            pltpu.SemaphoreType.DMA((2, G_RS)),
            pltpu.SemaphoreType.DMA((2, G_RS)),
            pltpu.SemaphoreType.DMA((2, G_RS)),
            pltpu.SemaphoreType.DMA((3, 4)),
            pltpu.SemaphoreType.DMA((3, 4)),
            pltpu.SemaphoreType.DMA((3, 4)),
            pltpu.SemaphoreType.DMA((3, 4)),
            pltpu.SemaphoreType.DMA((N_DEV,)),
            pltpu.SemaphoreType.DMA((N_DEV,)),
            pltpu.SemaphoreType.REGULAR((2, G_RS)),
            pltpu.SemaphoreType.REGULAR((2, G_RS)),
        ],
        compiler_params=pltpu.CompilerParams(collective_id=0),
    )(x, w_mat)
